# baseline (speedup 1.0000x reference)
import functools

import jax
import jax.numpy as jnp
from jax import lax
from jax.experimental import pallas as pl
from jax.experimental.pallas import tpu as pltpu
from jax.experimental.pallas import tpu_sc as plsc

K = 8192
D = 256
BETA = 0.25
TN = 256


_CHUNKS = ((0, 2736), (2736, 5472), (5472, 8192))


def _bf16_rne(x):
    b = lax.bitcast_convert_type(x, jnp.uint32)
    r = (b + jnp.uint32(0x7FFF) + ((b >> 16) & jnp.uint32(1))) & jnp.uint32(0xFFFF0000)
    return lax.bitcast_convert_type(r, jnp.float32)


def _w2_body(cb_ref, w2_ref):
    w = cb_ref[...]
    w2_ref[...] = jnp.sum(w * w, axis=1)


def _w2_kernel(codebook):
    return pl.pallas_call(
        _w2_body,
        out_shape=jax.ShapeDtypeStruct((K,), jnp.float32),
    )(codebook)


def _argmin_body(flat_ref, cb_ref, w2_ref, idx_ref, loss_ref, acc_ref):
    i = pl.program_id(0)
    f = flat_ref[...]
    w = cb_ref[...]
    x2 = jnp.sum(f * f, axis=1, keepdims=True)
    w2 = w2_ref[...]
    ab2 = lax.dot_general(f + f, w, (((1,), (1,)), ((), ())),
                          preferred_element_type=jnp.float32)
    dist = (x2 + w2[None, :]) - ab2

    def chunk_fast(c0, c1):
        dd = dist[:, c0:c1]
        m = jnp.min(dd, axis=1)
        match = jnp.where(dd <= m[:, None], 1.0, 0.0)
        colid = lax.broadcasted_iota(jnp.int32, (c1 - c0, 2), 1)
        rowv = lax.broadcasted_iota(jnp.int32, (c1 - c0, 2), 0).astype(jnp.float32)
        rhs = jnp.where(colid == 0, 1.0, rowv)
        s = lax.dot_general(match, rhs, (((1,), (0,)), ((), ())),
                            preferred_element_type=jnp.float32)
        cnt = s[:, 0]
        ci = s[:, 1].astype(jnp.int32) + c0
        return m, ci, cnt

    def chunk_slow(c0, c1, m):
        dd = dist[:, c0:c1]
        cols = lax.broadcasted_iota(jnp.int32, dd.shape, 1) + c0
        return jnp.min(jnp.where(dd <= m[:, None], cols, jnp.int32(K)), axis=1)

    ms, fast_is, cnts, takes = [], [], [], []
    m0, i0, cnt0 = chunk_fast(*_CHUNKS[0])
    ms.append(m0)
    fast_is.append(i0)
    cnts.append(cnt0)
    idx = i0
    md = m0
    qv = _bf16_rne(m0)
    for c0, c1 in _CHUNKS[1:]:
        mc, ic, cntc = chunk_fast(c0, c1)
        ms.append(mc)
        cnts.append(cntc)
        take = mc < qv
        takes.append(take)
        idx = jnp.where(take, ic, idx)
        md = jnp.where(take, mc, md)
        qv = _bf16_rne(jnp.where(take, mc, qv))
    idx_ref[...] = idx

    bad = (jnp.max(cnts[0]) > 1.0) | (jnp.max(cnts[1]) > 1.0) | (jnp.max(cnts[2]) > 1.0)

    @pl.when(bad)
    def _():
        s0 = chunk_slow(*_CHUNKS[0], ms[0])
        s1 = chunk_slow(*_CHUNKS[1], ms[1])
        s2 = chunk_slow(*_CHUNKS[2], ms[2])
        sidx = jnp.where(takes[0], s1, s0)
        sidx = jnp.where(takes[1], s2, sidx)
        idx_ref[...] = sidx

    @pl.when(i == 0)
    def _():
        acc_ref[0] = 0.0

    acc_ref[0] += jnp.sum(md)

    @pl.when(i == pl.num_programs(0) - 1)
    def _():
        scale = BETA / (flat_ref.shape[0] * pl.num_programs(0) * D)
        loss_ref[...] = jnp.full((1, 1), acc_ref[0] * scale, jnp.float32)


def _argmin_tc(flat, codebook):
    n = flat.shape[0]
    grid = (n // TN,)
    w2 = _w2_kernel(codebook)
    return pl.pallas_call(
        _argmin_body,
        grid=grid,
        in_specs=[
            pl.BlockSpec((TN, D), lambda i: (i, 0)),
            pl.BlockSpec((K, D), lambda i: (0, 0)),
            pl.BlockSpec((K,), lambda i: (0,)),
        ],
        out_specs=[
            pl.BlockSpec((TN,), lambda i: (i,)),
            pl.BlockSpec((1, 1), lambda i: (0, 0)),
        ],
        out_shape=[
            jax.ShapeDtypeStruct((n,), jnp.int32),
            jax.ShapeDtypeStruct((1, 1), jnp.float32),
        ],
        scratch_shapes=[pltpu.SMEM((1,), jnp.float32)],
        compiler_params=pltpu.CompilerParams(
            dimension_semantics=("arbitrary",),
        ),
    )(flat, codebook, w2)


def _gather_sc(codebook, idx):
    info = plsc.get_sparse_core_info()
    nc, ns = info.num_cores, info.num_subcores
    nw = nc * ns
    n = idx.shape[0]
    b_per_w = n // nw
    chunk = 128
    nchunk = b_per_w // chunk
    mesh = plsc.VectorSubcoreMesh(core_axis_name="c", subcore_axis_name="s")

    @functools.partial(
        pl.kernel,
        out_type=jax.ShapeDtypeStruct((n, D), jnp.float32),
        mesh=mesh,
        scratch_types=[
            pltpu.VMEM((chunk,), jnp.int32),
            pltpu.VMEM((chunk, D), jnp.float32),
            pltpu.SemaphoreType.DMA,
        ],
    )
    def k(cb_hbm, idx_hbm, out_hbm, idx_v, rows_v, sem):
        wid = lax.axis_index("s") * nc + lax.axis_index("c")
        base = wid * b_per_w
        for c in range(nchunk):
            pltpu.sync_copy(idx_hbm.at[pl.ds(base + c * chunk, chunk)], idx_v)
            pltpu.async_copy(cb_hbm.at[idx_v], rows_v, sem).wait()
            pltpu.sync_copy(rows_v, out_hbm.at[pl.ds(base + c * chunk, chunk)])

    return k(codebook, idx)


def kernel(z, codebook):
    b, c, h, w = z.shape
    z_nhwc = jnp.transpose(z, (0, 2, 3, 1))
    flat = z_nhwc.reshape(-1, D)
    idx, loss = _argmin_tc(flat, codebook)
    zq = _gather_sc(codebook, idx)
    z_q_out = jnp.transpose(zq.reshape(b, h, w, c), (0, 3, 1, 2))
    return (z_q_out, idx, loss.reshape(()))


# speedup vs baseline: 1.1317x; 1.1317x over previous
import functools

import jax
import jax.numpy as jnp
from jax import lax
from jax.experimental import pallas as pl
from jax.experimental.pallas import tpu as pltpu
from jax.experimental.pallas import tpu_sc as plsc

K = 8192
D = 256
BETA = 0.25
TN = 512


_CHUNKS = ((0, 2736), (2736, 5472), (5472, 8192))


def _bf16_rne(x):
    b = lax.bitcast_convert_type(x, jnp.uint32)
    r = (b + jnp.uint32(0x7FFF) + ((b >> 16) & jnp.uint32(1))) & jnp.uint32(0xFFFF0000)
    return lax.bitcast_convert_type(r, jnp.float32)


def _w2_body(cb_ref, w2_ref):
    w = cb_ref[...]
    w2_ref[...] = jnp.sum(w * w, axis=1)


def _w2_kernel(codebook):
    return pl.pallas_call(
        _w2_body,
        out_shape=jax.ShapeDtypeStruct((K,), jnp.float32),
    )(codebook)


def _argmin_body(flat_ref, cb_ref, w2_ref, idx_ref, loss_ref, acc_ref):
    i = pl.program_id(0)
    f = flat_ref[...]
    w = cb_ref[...]
    x2 = jnp.sum(f * f, axis=1, keepdims=True)
    w2 = w2_ref[...]
    ab2 = lax.dot_general(f + f, w, (((1,), (1,)), ((), ())),
                          preferred_element_type=jnp.float32)
    dist = (x2 + w2[None, :]) - ab2

    def chunk_min(c0, c1):
        dd = dist[:, c0:c1]
        m = jnp.min(dd, axis=1)
        cols = lax.broadcasted_iota(jnp.int32, dd.shape, 1) + c0
        ci = jnp.min(jnp.where(dd <= m[:, None], cols, jnp.int32(K)), axis=1)
        return m, ci

    m0, idx = chunk_min(*_CHUNKS[0])
    md = m0
    qv = _bf16_rne(m0)
    for c0, c1 in _CHUNKS[1:]:
        mc, ic = chunk_min(c0, c1)
        take = mc < qv
        idx = jnp.where(take, ic, idx)
        md = jnp.where(take, mc, md)
        qv = _bf16_rne(jnp.where(take, mc, qv))
    idx_ref[...] = idx

    @pl.when(i == 0)
    def _():
        acc_ref[0] = 0.0

    acc_ref[0] += jnp.sum(md)

    @pl.when(i == pl.num_programs(0) - 1)
    def _():
        scale = BETA / (flat_ref.shape[0] * pl.num_programs(0) * D)
        loss_ref[...] = jnp.full((1, 1), acc_ref[0] * scale, jnp.float32)


def _argmin_tc(flat, codebook):
    n = flat.shape[0]
    grid = (n // TN,)
    w2 = _w2_kernel(codebook)
    return pl.pallas_call(
        _argmin_body,
        grid=grid,
        in_specs=[
            pl.BlockSpec((TN, D), lambda i: (i, 0)),
            pl.BlockSpec((K, D), lambda i: (0, 0)),
            pl.BlockSpec((K,), lambda i: (0,)),
        ],
        out_specs=[
            pl.BlockSpec((TN,), lambda i: (i,)),
            pl.BlockSpec((1, 1), lambda i: (0, 0)),
        ],
        out_shape=[
            jax.ShapeDtypeStruct((n,), jnp.int32),
            jax.ShapeDtypeStruct((1, 1), jnp.float32),
        ],
        scratch_shapes=[pltpu.SMEM((1,), jnp.float32)],
        compiler_params=pltpu.CompilerParams(
            dimension_semantics=("arbitrary",),
        ),
    )(flat, codebook, w2)


def _gather_sc(codebook, idx):
    info = plsc.get_sparse_core_info()
    nc, ns = info.num_cores, info.num_subcores
    nw = nc * ns
    n = idx.shape[0]
    b_per_w = n // nw
    chunk = 128
    nchunk = b_per_w // chunk
    mesh = plsc.VectorSubcoreMesh(core_axis_name="c", subcore_axis_name="s")

    @functools.partial(
        pl.kernel,
        out_type=jax.ShapeDtypeStruct((n, D), jnp.float32),
        mesh=mesh,
        scratch_types=[
            pltpu.VMEM((chunk,), jnp.int32),
            pltpu.VMEM((chunk, D), jnp.float32),
            pltpu.SemaphoreType.DMA,
        ],
    )
    def k(cb_hbm, idx_hbm, out_hbm, idx_v, rows_v, sem):
        wid = lax.axis_index("s") * nc + lax.axis_index("c")
        base = wid * b_per_w
        for c in range(nchunk):
            pltpu.sync_copy(idx_hbm.at[pl.ds(base + c * chunk, chunk)], idx_v)
            pltpu.async_copy(cb_hbm.at[idx_v], rows_v, sem).wait()
            pltpu.sync_copy(rows_v, out_hbm.at[pl.ds(base + c * chunk, chunk)])

    return k(codebook, idx)


def kernel(z, codebook):
    b, c, h, w = z.shape
    z_nhwc = jnp.transpose(z, (0, 2, 3, 1))
    flat = z_nhwc.reshape(-1, D)
    idx, loss = _argmin_tc(flat, codebook)
    zq = _gather_sc(codebook, idx)
    z_q_out = jnp.transpose(zq.reshape(b, h, w, c), (0, 3, 1, 2))
    return (z_q_out, idx, loss.reshape(()))


# speedup vs baseline: 1.1937x; 1.0548x over previous
import functools

import jax
import jax.numpy as jnp
from jax import lax
from jax.experimental import pallas as pl
from jax.experimental.pallas import tpu as pltpu
from jax.experimental.pallas import tpu_sc as plsc

K = 8192
D = 256
BETA = 0.25
TN = 1024


_CHUNKS = ((0, 2736), (2736, 5472), (5472, 8192))


def _bf16_rne(x):
    b = lax.bitcast_convert_type(x, jnp.uint32)
    r = (b + jnp.uint32(0x7FFF) + ((b >> 16) & jnp.uint32(1))) & jnp.uint32(0xFFFF0000)
    return lax.bitcast_convert_type(r, jnp.float32)


def _w2_body(cb_ref, w2_ref):
    w = cb_ref[...]
    w2_ref[...] = jnp.sum(w * w, axis=1)


def _w2_kernel(codebook):
    return pl.pallas_call(
        _w2_body,
        out_shape=jax.ShapeDtypeStruct((K,), jnp.float32),
    )(codebook)


def _argmin_body(flat_ref, cb_ref, w2_ref, idx_ref, loss_ref, acc_ref):
    i = pl.program_id(0)
    f = flat_ref[...]
    w = cb_ref[...]
    x2 = jnp.sum(f * f, axis=1, keepdims=True)
    w2 = w2_ref[...]
    ab2 = lax.dot_general(f + f, w, (((1,), (1,)), ((), ())),
                          preferred_element_type=jnp.float32)
    dist = (x2 + w2[None, :]) - ab2

    def chunk_min(c0, c1):
        dd = dist[:, c0:c1]
        m = jnp.min(dd, axis=1)
        cols = lax.broadcasted_iota(jnp.int32, dd.shape, 1) + c0
        ci = jnp.min(jnp.where(dd <= m[:, None], cols, jnp.int32(K)), axis=1)
        return m, ci

    m0, idx = chunk_min(*_CHUNKS[0])
    md = m0
    qv = _bf16_rne(m0)
    for c0, c1 in _CHUNKS[1:]:
        mc, ic = chunk_min(c0, c1)
        take = mc < qv
        idx = jnp.where(take, ic, idx)
        md = jnp.where(take, mc, md)
        qv = _bf16_rne(jnp.where(take, mc, qv))
    idx_ref[...] = idx

    @pl.when(i == 0)
    def _():
        acc_ref[0] = 0.0

    acc_ref[0] += jnp.sum(md)

    @pl.when(i == pl.num_programs(0) - 1)
    def _():
        scale = BETA / (flat_ref.shape[0] * pl.num_programs(0) * D)
        loss_ref[...] = jnp.full((1, 1), acc_ref[0] * scale, jnp.float32)


def _argmin_tc(flat, codebook):
    n = flat.shape[0]
    grid = (n // TN,)
    w2 = _w2_kernel(codebook)
    return pl.pallas_call(
        _argmin_body,
        grid=grid,
        in_specs=[
            pl.BlockSpec((TN, D), lambda i: (i, 0)),
            pl.BlockSpec((K, D), lambda i: (0, 0)),
            pl.BlockSpec((K,), lambda i: (0,)),
        ],
        out_specs=[
            pl.BlockSpec((TN,), lambda i: (i,)),
            pl.BlockSpec((1, 1), lambda i: (0, 0)),
        ],
        out_shape=[
            jax.ShapeDtypeStruct((n,), jnp.int32),
            jax.ShapeDtypeStruct((1, 1), jnp.float32),
        ],
        scratch_shapes=[pltpu.SMEM((1,), jnp.float32)],
        compiler_params=pltpu.CompilerParams(
            dimension_semantics=("arbitrary",),
        ),
    )(flat, codebook, w2)


def _gather_sc(codebook, idx):
    info = plsc.get_sparse_core_info()
    nc, ns = info.num_cores, info.num_subcores
    nw = nc * ns
    n = idx.shape[0]
    b_per_w = n // nw
    chunk = 128
    nchunk = b_per_w // chunk
    mesh = plsc.VectorSubcoreMesh(core_axis_name="c", subcore_axis_name="s")

    @functools.partial(
        pl.kernel,
        out_type=jax.ShapeDtypeStruct((n, D), jnp.float32),
        mesh=mesh,
        scratch_types=[
            pltpu.VMEM((chunk,), jnp.int32),
            pltpu.VMEM((chunk, D), jnp.float32),
            pltpu.SemaphoreType.DMA,
        ],
    )
    def k(cb_hbm, idx_hbm, out_hbm, idx_v, rows_v, sem):
        wid = lax.axis_index("s") * nc + lax.axis_index("c")
        base = wid * b_per_w
        for c in range(nchunk):
            pltpu.sync_copy(idx_hbm.at[pl.ds(base + c * chunk, chunk)], idx_v)
            pltpu.async_copy(cb_hbm.at[idx_v], rows_v, sem).wait()
            pltpu.sync_copy(rows_v, out_hbm.at[pl.ds(base + c * chunk, chunk)])

    return k(codebook, idx)


def kernel(z, codebook):
    b, c, h, w = z.shape
    z_nhwc = jnp.transpose(z, (0, 2, 3, 1))
    flat = z_nhwc.reshape(-1, D)
    idx, loss = _argmin_tc(flat, codebook)
    zq = _gather_sc(codebook, idx)
    z_q_out = jnp.transpose(zq.reshape(b, h, w, c), (0, 3, 1, 2))
    return (z_q_out, idx, loss.reshape(()))


# speedup vs baseline: 1.2666x; 1.0611x over previous
import functools

import jax
import jax.numpy as jnp
from jax import lax
from jax.experimental import pallas as pl
from jax.experimental.pallas import tpu as pltpu
from jax.experimental.pallas import tpu_sc as plsc

K = 8192
D = 256
BETA = 0.25
TN = 2048


_CHUNKS = ((0, 2736), (2736, 5472), (5472, 8192))


def _bf16_rne(x):
    b = lax.bitcast_convert_type(x, jnp.uint32)
    r = (b + jnp.uint32(0x7FFF) + ((b >> 16) & jnp.uint32(1))) & jnp.uint32(0xFFFF0000)
    return lax.bitcast_convert_type(r, jnp.float32)


def _w2_body(cb_ref, w2_ref):
    w = cb_ref[...]
    w2_ref[...] = jnp.sum(w * w, axis=1)


def _w2_kernel(codebook):
    return pl.pallas_call(
        _w2_body,
        out_shape=jax.ShapeDtypeStruct((K,), jnp.float32),
    )(codebook)


def _argmin_body(flat_ref, cb_ref, w2_ref, idx_ref, loss_ref, acc_ref):
    i = pl.program_id(0)
    f = flat_ref[...]
    w = cb_ref[...]
    x2 = jnp.sum(f * f, axis=1, keepdims=True)
    w2 = w2_ref[...]
    ab2 = lax.dot_general(f + f, w, (((1,), (1,)), ((), ())),
                          preferred_element_type=jnp.float32)
    dist = (x2 + w2[None, :]) - ab2

    def chunk_min(c0, c1):
        dd = dist[:, c0:c1]
        m = jnp.min(dd, axis=1)
        cols = lax.broadcasted_iota(jnp.int32, dd.shape, 1) + c0
        ci = jnp.min(jnp.where(dd <= m[:, None], cols, jnp.int32(K)), axis=1)
        return m, ci

    m0, idx = chunk_min(*_CHUNKS[0])
    md = m0
    qv = _bf16_rne(m0)
    for c0, c1 in _CHUNKS[1:]:
        mc, ic = chunk_min(c0, c1)
        take = mc < qv
        idx = jnp.where(take, ic, idx)
        md = jnp.where(take, mc, md)
        qv = _bf16_rne(jnp.where(take, mc, qv))
    idx_ref[...] = idx

    @pl.when(i == 0)
    def _():
        acc_ref[0] = 0.0

    acc_ref[0] += jnp.sum(md)

    @pl.when(i == pl.num_programs(0) - 1)
    def _():
        scale = BETA / (flat_ref.shape[0] * pl.num_programs(0) * D)
        loss_ref[...] = jnp.full((1, 1), acc_ref[0] * scale, jnp.float32)


def _argmin_tc(flat, codebook):
    n = flat.shape[0]
    grid = (n // TN,)
    w2 = _w2_kernel(codebook)
    return pl.pallas_call(
        _argmin_body,
        grid=grid,
        in_specs=[
            pl.BlockSpec((TN, D), lambda i: (i, 0)),
            pl.BlockSpec((K, D), lambda i: (0, 0)),
            pl.BlockSpec((K,), lambda i: (0,)),
        ],
        out_specs=[
            pl.BlockSpec((TN,), lambda i: (i,)),
            pl.BlockSpec((1, 1), lambda i: (0, 0)),
        ],
        out_shape=[
            jax.ShapeDtypeStruct((n,), jnp.int32),
            jax.ShapeDtypeStruct((1, 1), jnp.float32),
        ],
        scratch_shapes=[pltpu.SMEM((1,), jnp.float32)],
        compiler_params=pltpu.CompilerParams(
            dimension_semantics=("arbitrary",),
        ),
    )(flat, codebook, w2)


def _gather_sc(codebook, idx):
    info = plsc.get_sparse_core_info()
    nc, ns = info.num_cores, info.num_subcores
    nw = nc * ns
    n = idx.shape[0]
    b_per_w = n // nw
    chunk = 128
    nchunk = b_per_w // chunk
    mesh = plsc.VectorSubcoreMesh(core_axis_name="c", subcore_axis_name="s")

    @functools.partial(
        pl.kernel,
        out_type=jax.ShapeDtypeStruct((n, D), jnp.float32),
        mesh=mesh,
        scratch_types=[
            pltpu.VMEM((chunk,), jnp.int32),
            pltpu.VMEM((chunk, D), jnp.float32),
            pltpu.SemaphoreType.DMA,
        ],
    )
    def k(cb_hbm, idx_hbm, out_hbm, idx_v, rows_v, sem):
        wid = lax.axis_index("s") * nc + lax.axis_index("c")
        base = wid * b_per_w
        for c in range(nchunk):
            pltpu.sync_copy(idx_hbm.at[pl.ds(base + c * chunk, chunk)], idx_v)
            pltpu.async_copy(cb_hbm.at[idx_v], rows_v, sem).wait()
            pltpu.sync_copy(rows_v, out_hbm.at[pl.ds(base + c * chunk, chunk)])

    return k(codebook, idx)


def kernel(z, codebook):
    b, c, h, w = z.shape
    z_nhwc = jnp.transpose(z, (0, 2, 3, 1))
    flat = z_nhwc.reshape(-1, D)
    idx, loss = _argmin_tc(flat, codebook)
    zq = _gather_sc(codebook, idx)
    z_q_out = jnp.transpose(zq.reshape(b, h, w, c), (0, 3, 1, 2))
    return (z_q_out, idx, loss.reshape(()))
